# Initial kernel scaffold; baseline (speedup 1.0000x reference)
import functools

import jax
import jax.numpy as jnp
import numpy as np
from jax import lax
from jax.experimental import pallas as pl
from jax.experimental.pallas import tpu as pltpu
from jax.experimental.pallas import tpu_sc as plsc

N = 10000
E = 320000
H = 128
NF = 128
NG = 50
B = 100
L = 3
CUTOFF = 10.0
PI = float(np.pi)

NPAD = 10240
NW = 32
EPW = E // NW
ECH = 80
NCHUNKS = EPW // ECH
NPW = NPAD // NW
NROWS_PER_SUB = NPAD // 16

_offset_np = np.linspace(0.0, CUTOFF, NG).astype(np.float32)
_COEFF = float(-0.5 / (_offset_np[1] - _offset_np[0]) ** 2)
_SHIFT = float(np.log(2.0))


def _ssp(x):
    return jnp.maximum(x, 0.0) + jnp.log1p(jnp.exp(-jnp.abs(x))) - _SHIFT


_MESH = plsc.VectorSubcoreMesh(core_axis_name="c", subcore_axis_name="s")


def _prep_body(z_hbm, pos_hbm, row_hbm, col_hbm, table_hbm,
               v0_hbm, d2_hbm,
               z_v, rows_v, pos_v, ridx_v, cidx_v, d2_v, sem):
    c = lax.axis_index("c")
    s = lax.axis_index("s")
    wid = s * 2 + c

    nbase = wid * NPW
    pltpu.sync_copy(z_hbm.at[pl.ds(nbase, NPW)], z_v)
    for ci in range(NPW // ECH):
        idx = z_v.at[pl.ds(ci * ECH, ECH)]
        pltpu.async_copy(table_hbm.at[idx], rows_v, sem).wait()
        pltpu.sync_copy(rows_v, v0_hbm.at[pl.ds(nbase + ci * ECH, ECH)])

    ebase = wid * EPW
    pltpu.sync_copy(pos_hbm, pos_v)
    pltpu.sync_copy(row_hbm.at[pl.ds(ebase, EPW)], ridx_v)
    pltpu.sync_copy(col_hbm.at[pl.ds(ebase, EPW)], cidx_v)

    def body(i, carry):
        r = ridx_v[pl.ds(i * 16, 16)]
        cc = cidx_v[pl.ds(i * 16, 16)]
        acc = jnp.zeros((16,), jnp.float32)
        for j in range(3):
            jv = jnp.full((16,), j, jnp.int32)
            pr = plsc.load_gather(pos_v, [r, jv])
            pc = plsc.load_gather(pos_v, [cc, jv])
            d = pr - pc
            acc = acc + d * d
        d2_v[pl.ds(i * 16, 16)] = acc
        return carry

    lax.fori_loop(0, EPW // 16, body, 0)
    pltpu.sync_copy(d2_v, d2_hbm.at[pl.ds(ebase, EPW)])


_prep = functools.partial(
    pl.kernel,
    out_type=(jax.ShapeDtypeStruct((NPAD, H), jnp.float32),
              jax.ShapeDtypeStruct((E,), jnp.float32)),
    mesh=_MESH,
    scratch_types=[
        pltpu.VMEM((NPW,), jnp.int32),
        pltpu.VMEM((ECH, H), jnp.float32),
        pltpu.VMEM((N, 3), jnp.float32),
        pltpu.VMEM((EPW,), jnp.int32),
        pltpu.VMEM((EPW,), jnp.int32),
        pltpu.VMEM((EPW,), jnp.float32),
        pltpu.SemaphoreType.DMA,
    ],
)(_prep_body)


def _edge_body(vv_hbm, w_hbm, row_hbm, col_hbm,
               out_hbm,
               acc_sm, ridx_v, cidx_v, vvr_v, w_v, sem):
    c = lax.axis_index("c")
    s = lax.axis_index("s")
    wid = s * 2 + c

    zero16 = jnp.zeros((16,), jnp.float32)

    def zbody(e, carry):
        for j in range(H // 16):
            vvr_v[e, pl.ds(j * 16, 16)] = zero16
        return carry

    lax.fori_loop(0, ECH, zbody, 0)
    for b in range(NROWS_PER_SUB // ECH):
        pltpu.sync_copy(vvr_v, acc_sm.at[pl.ds(s * NROWS_PER_SUB + b * ECH, ECH)])
    plsc.subcore_barrier()

    def body(i, carry):
        base = wid * EPW + i * ECH
        pltpu.sync_copy(row_hbm.at[pl.ds(base, ECH)], ridx_v)
        pltpu.sync_copy(col_hbm.at[pl.ds(base, ECH)], cidx_v)
        pltpu.sync_copy(w_hbm.at[pl.ds(base, ECH)], w_v)
        pltpu.async_copy(vv_hbm.at[ridx_v], vvr_v, sem).wait()

        def mbody(e, carry2):
            for j in range(H // 16):
                sl = pl.ds(j * 16, 16)
                w_v[e, sl] = w_v[e, sl] * vvr_v[e, sl]
            return carry2

        lax.fori_loop(0, ECH, mbody, 0)
        pltpu.sync_copy(w_v, acc_sm.at[cidx_v], add=True)
        return carry

    lax.fori_loop(0, NCHUNKS, body, 0)
    plsc.subcore_barrier()
    pltpu.sync_copy(acc_sm.at[pl.ds(s * NROWS_PER_SUB, NROWS_PER_SUB)],
                    out_hbm.at[pl.ds(c * NPAD + s * NROWS_PER_SUB, NROWS_PER_SUB)])


_edge = functools.partial(
    pl.kernel,
    out_type=jax.ShapeDtypeStruct((2 * NPAD, H), jnp.float32),
    mesh=_MESH,
    scratch_types=[
        pltpu.VMEM_SHARED((NPAD, H), jnp.float32),
        pltpu.VMEM((ECH,), jnp.int32),
        pltpu.VMEM((ECH,), jnp.int32),
        pltpu.VMEM((ECH, H), jnp.float32),
        pltpu.VMEM((ECH, H), jnp.float32),
        pltpu.SemaphoreType.DMA,
    ],
)(_edge_body)


EB = 512


def _filter_body(d2_ref, offs_ref,
                 w00, b00, w10, b10,
                 w01, b01, w11, b11,
                 w02, b02, w12, b12,
                 o0, o1, o2):
    d2 = d2_ref[...]
    dist = jnp.sqrt(d2)
    delta = dist - offs_ref[...]
    demb = jnp.exp(_COEFF * delta * delta)
    cc = 0.5 * (jnp.cos(dist * (PI / CUTOFF)) + 1.0)
    for w0r, b0r, w1r, b1r, o in ((w00, b00, w10, b10, o0),
                                  (w01, b01, w11, b11, o1),
                                  (w02, b02, w12, b12, o2)):
        h = jnp.dot(demb, w0r[...], preferred_element_type=jnp.float32) + b0r[...]
        h = _ssp(h)
        w = jnp.dot(h, w1r[...], preferred_element_type=jnp.float32) + b1r[...]
        o[...] = w * cc


def _filter(d2c, offs, layer_ws):
    full = lambda shape: pl.BlockSpec(shape, lambda i: (0, 0))
    in_specs = [pl.BlockSpec((EB, 1), lambda i: (i, 0)), full((1, H))]
    args = [d2c, offs]
    for (w0p, b0, w1, b1) in layer_ws:
        in_specs += [full((H, H)), full((1, H)), full((H, H)), full((1, H))]
        args += [w0p, b0, w1, b1]
    out_sd = jax.ShapeDtypeStruct((E, H), jnp.float32)
    return pl.pallas_call(
        _filter_body,
        grid=(E // EB,),
        in_specs=in_specs,
        out_specs=[pl.BlockSpec((EB, H), lambda i: (i, 0))] * 3,
        out_shape=[out_sd] * 3,
    )(*args)


NB = 512


def _mm_body(x_ref, w_ref, o_ref):
    o_ref[...] = jnp.dot(x_ref[...], w_ref[...],
                         preferred_element_type=jnp.float32)


def _matmul(x, w):
    return pl.pallas_call(
        _mm_body,
        grid=(NPAD // NB,),
        in_specs=[pl.BlockSpec((NB, H), lambda i: (i, 0)),
                  pl.BlockSpec((H, H), lambda i: (0, 0))],
        out_specs=pl.BlockSpec((NB, H), lambda i: (i, 0)),
        out_shape=jax.ShapeDtypeStruct((NPAD, H), jnp.float32),
    )(x, w)


def _node_body(p0_ref, p1_ref, v_ref, w1_ref, b1_ref, w2_ref, b2_ref, o_ref):
    agg = p0_ref[...] + p1_ref[...]
    t = _ssp(jnp.dot(agg, w1_ref[...], preferred_element_type=jnp.float32)
             + b1_ref[...])
    o_ref[...] = (v_ref[...]
                  + jnp.dot(t, w2_ref[...], preferred_element_type=jnp.float32)
                  + b2_ref[...])


def _node(p0, p1, v, w1, b1, w2, b2):
    blk = pl.BlockSpec((NB, H), lambda i: (i, 0))
    full = lambda shape: pl.BlockSpec(shape, lambda i: (0, 0))
    return pl.pallas_call(
        _node_body,
        grid=(NPAD // NB,),
        in_specs=[blk, blk, blk, full((H, H)), full((1, H)),
                  full((H, H)), full((1, H))],
        out_specs=blk,
        out_shape=jax.ShapeDtypeStruct((NPAD, H), jnp.float32),
    )(p0, p1, v, w1, b1, w2, b2)


def _final_body(v_ref, batch_ref, u1_ref, b1_ref, u2_ref, b2_ref, o_ref):
    i = pl.program_id(0)

    @pl.when(i == 0)
    def _():
        o_ref[...] = jnp.zeros_like(o_ref)

    t = _ssp(jnp.dot(v_ref[...], u1_ref[...],
                     preferred_element_type=jnp.float32) + b1_ref[...])
    h = jnp.dot(t, u2_ref[...], preferred_element_type=jnp.float32) + b2_ref[...]
    rows = lax.broadcasted_iota(jnp.int32, (H, 1), 0)
    obt = (rows == batch_ref[...]).astype(jnp.float32)
    o_ref[...] += jnp.dot(obt, h, preferred_element_type=jnp.float32)


def _final(v, batch_row, u1p, b1p, u2p, b2r):
    full = lambda shape: pl.BlockSpec(shape, lambda i: (0, 0))
    return pl.pallas_call(
        _final_body,
        grid=(NPAD // NB,),
        in_specs=[pl.BlockSpec((NB, H), lambda i: (i, 0)),
                  pl.BlockSpec((1, NB), lambda i: (0, i)),
                  full((H, H)), full((1, H)), full((H, H)), full((1, H))],
        out_specs=full((H, H)),
        out_shape=jax.ShapeDtypeStruct((H, H), jnp.float32),
    )(v, batch_row, u1p, b1p, u2p, b2r)


def kernel(z, pos, batch, edge_index, params):
    row = edge_index[0].astype(jnp.int32)
    col = edge_index[1].astype(jnp.int32)
    zp = jnp.concatenate([z.astype(jnp.int32),
                          jnp.zeros((NPAD - N,), jnp.int32)])
    batchp = jnp.concatenate([batch.astype(jnp.int32),
                              jnp.full((NPAD - N,), H - 1, jnp.int32)])
    batch_row = batchp.reshape(1, NPAD)

    v0, d2 = _prep(zp, pos.astype(jnp.float32), row, col,
                   params['emb_table'].astype(jnp.float32))
    d2c = d2.reshape(E, 1)

    offs = jnp.zeros((1, H), jnp.float32).at[0, :NG].set(
        jnp.asarray(_offset_np))
    layer_ws = []
    for l in range(L):
        p = params['layers'][l]
        w0p = jnp.zeros((H, H), jnp.float32).at[:NG, :].set(p['e_mlp_W0'])
        layer_ws.append((w0p, p['e_mlp_b0'].reshape(1, H),
                         p['e_mlp_W1'], p['e_mlp_b1'].reshape(1, H)))
    w_filters = _filter(d2c, offs, layer_ws)

    v = v0
    for l in range(L):
        p = params['layers'][l]
        vv = _matmul(v, p['e_lin_W'])
        parts = _edge(vv, w_filters[l], row, col)
        v = _node(parts[:NPAD], parts[NPAD:], v,
                  p['v_lin1_W'], p['v_lin1_b'].reshape(1, H),
                  p['v_lin2_W'], p['v_lin2_b'].reshape(1, H))

    u1p = jnp.zeros((H, H), jnp.float32).at[:, :H // 2].set(params['u_lin1_W'])
    b1p = jnp.zeros((1, H), jnp.float32).at[0, :H // 2].set(params['u_lin1_b'])
    u2p = jnp.zeros((H, H), jnp.float32).at[:H // 2, 0].set(
        params['u_lin2_W'][:, 0])
    b2r = jnp.broadcast_to(params['u_lin2_b'].reshape(1, 1), (1, H))

    u_full = _final(v, batch_row, u1p, b1p, u2p, b2r)
    u = u_full[:B, :1]
    return (u, 0.0, True, 69.0)


# speedup vs baseline: 2.2669x; 2.2669x over previous
import functools

import jax
import jax.numpy as jnp
import numpy as np
from jax import lax
from jax.experimental import pallas as pl
from jax.experimental.pallas import tpu as pltpu
from jax.experimental.pallas import tpu_sc as plsc

N = 10000
E = 320000
H = 128
NF = 128
NG = 50
B = 100
L = 3
CUTOFF = 10.0
PI = float(np.pi)

NPAD = 10240
NW = 32
EPW = E // NW
ECH = 80
NCHUNKS = EPW // ECH
NPW = NPAD // NW
NROWS_PER_SUB = NPAD // 16

_offset_np = np.linspace(0.0, CUTOFF, NG).astype(np.float32)
_COEFF = float(-0.5 / (_offset_np[1] - _offset_np[0]) ** 2)
_SHIFT = float(np.log(2.0))


def _ssp(x):
    return jnp.maximum(x, 0.0) + jnp.log1p(jnp.exp(-jnp.abs(x))) - _SHIFT


def _mesh():
    return plsc.VectorSubcoreMesh(core_axis_name="c", subcore_axis_name="s",
                                  num_cores=2, num_subcores=16)


def _prep_body(z_hbm, pos_hbm, row_hbm, col_hbm, table_hbm,
               v0_hbm, d2_hbm,
               z_v, rows_v, pos_v, ridx_v, cidx_v, d2_v, sem):
    c = lax.axis_index("c")
    s = lax.axis_index("s")
    wid = s * 2 + c

    nbase = wid * NPW
    pltpu.sync_copy(z_hbm.at[pl.ds(nbase, NPW)], z_v)
    for ci in range(NPW // ECH):
        idx = z_v.at[pl.ds(ci * ECH, ECH)]
        pltpu.async_copy(table_hbm.at[idx], rows_v, sem).wait()
        pltpu.sync_copy(rows_v, v0_hbm.at[pl.ds(nbase + ci * ECH, ECH)])

    ebase = wid * EPW
    pltpu.sync_copy(pos_hbm, pos_v)
    pltpu.sync_copy(row_hbm.at[pl.ds(ebase, EPW)], ridx_v)
    pltpu.sync_copy(col_hbm.at[pl.ds(ebase, EPW)], cidx_v)

    def body(i, carry):
        r3 = ridx_v[pl.ds(i * 16, 16)] * 3
        c3 = cidx_v[pl.ds(i * 16, 16)] * 3
        acc = jnp.zeros((16,), jnp.float32)
        for j in range(3):
            pr = plsc.load_gather(pos_v, [r3 + j])
            pc = plsc.load_gather(pos_v, [c3 + j])
            d = pr - pc
            acc = acc + d * d
        d2_v[pl.ds(i * 16, 16)] = acc
        return carry

    lax.fori_loop(0, EPW // 16, body, 0)
    pltpu.sync_copy(d2_v, d2_hbm.at[pl.ds(ebase, EPW)])


@functools.lru_cache(maxsize=None)
def _prep():
  return functools.partial(
    pl.kernel,
    out_type=(jax.ShapeDtypeStruct((NPAD, H), jnp.float32),
              jax.ShapeDtypeStruct((E,), jnp.float32)),
    mesh=_mesh(),
    scratch_types=[
        pltpu.VMEM((NPW,), jnp.int32),
        pltpu.VMEM((ECH, H), jnp.float32),
        pltpu.VMEM((N * 3,), jnp.float32),
        pltpu.VMEM((EPW,), jnp.int32),
        pltpu.VMEM((EPW,), jnp.int32),
        pltpu.VMEM((EPW,), jnp.float32),
        pltpu.SemaphoreType.DMA,
    ],
    compiler_params=pltpu.CompilerParams(needs_layout_passes=False),
  )(_prep_body)


def _edge_body(vv_hbm, w_hbm, row_hbm, col_hbm,
               out_hbm,
               acc_sm, ridx_v, cidx_v, vvr_v, w_v, sem):
    c = lax.axis_index("c")
    s = lax.axis_index("s")
    wid = s * 2 + c

    zero16 = jnp.zeros((16,), jnp.float32)

    def zbody(e, carry):
        for j in range(H // 16):
            vvr_v[e, pl.ds(j * 16, 16)] = zero16
        return carry

    lax.fori_loop(0, ECH, zbody, 0)
    for b in range(NROWS_PER_SUB // ECH):
        pltpu.sync_copy(vvr_v, acc_sm.at[pl.ds(s * NROWS_PER_SUB + b * ECH, ECH)])
    plsc.subcore_barrier()

    def body(i, carry):
        base = wid * EPW + i * ECH
        pltpu.sync_copy(row_hbm.at[pl.ds(base, ECH)], ridx_v)
        pltpu.sync_copy(col_hbm.at[pl.ds(base, ECH)], cidx_v)
        pltpu.sync_copy(w_hbm.at[pl.ds(base, ECH)], w_v)
        pltpu.async_copy(vv_hbm.at[ridx_v], vvr_v, sem).wait()

        def mbody(e, carry2):
            for j in range(H // 16):
                sl = pl.ds(j * 16, 16)
                w_v[e, sl] = w_v[e, sl] * vvr_v[e, sl]
            return carry2

        lax.fori_loop(0, ECH, mbody, 0)
        pltpu.sync_copy(w_v, acc_sm.at[cidx_v], add=True)
        return carry

    lax.fori_loop(0, NCHUNKS, body, 0)
    plsc.subcore_barrier()
    pltpu.sync_copy(acc_sm.at[pl.ds(s * NROWS_PER_SUB, NROWS_PER_SUB)],
                    out_hbm.at[pl.ds(c * NPAD + s * NROWS_PER_SUB, NROWS_PER_SUB)])


@functools.lru_cache(maxsize=None)
def _edge():
  return functools.partial(
    pl.kernel,
    out_type=jax.ShapeDtypeStruct((2 * NPAD, H), jnp.float32),
    mesh=_mesh(),
    scratch_types=[
        pltpu.VMEM_SHARED((NPAD, H), jnp.float32),
        pltpu.VMEM((ECH,), jnp.int32),
        pltpu.VMEM((ECH,), jnp.int32),
        pltpu.VMEM((ECH, H), jnp.float32),
        pltpu.VMEM((ECH, H), jnp.float32),
        pltpu.SemaphoreType.DMA,
    ],
    compiler_params=pltpu.CompilerParams(needs_layout_passes=False),
  )(_edge_body)


EB = 512


def _filter_body(d2_ref, offs_ref,
                 w00, b00, w10, b10,
                 w01, b01, w11, b11,
                 w02, b02, w12, b12,
                 o0, o1, o2):
    d2 = d2_ref[...]
    dist = jnp.sqrt(d2)
    delta = dist - offs_ref[...]
    demb = jnp.exp(_COEFF * delta * delta)
    cc = 0.5 * (jnp.cos(dist * (PI / CUTOFF)) + 1.0)
    for w0r, b0r, w1r, b1r, o in ((w00, b00, w10, b10, o0),
                                  (w01, b01, w11, b11, o1),
                                  (w02, b02, w12, b12, o2)):
        h = jnp.dot(demb, w0r[...], preferred_element_type=jnp.float32) + b0r[...]
        h = _ssp(h)
        w = jnp.dot(h, w1r[...], preferred_element_type=jnp.float32) + b1r[...]
        o[...] = w * cc


def _filter(d2c, offs, layer_ws):
    full = lambda shape: pl.BlockSpec(shape, lambda i: (0, 0))
    in_specs = [pl.BlockSpec((EB, 1), lambda i: (i, 0)), full((1, H))]
    args = [d2c, offs]
    for (w0p, b0, w1, b1) in layer_ws:
        in_specs += [full((H, H)), full((1, H)), full((H, H)), full((1, H))]
        args += [w0p, b0, w1, b1]
    out_sd = jax.ShapeDtypeStruct((E, H), jnp.float32)
    return pl.pallas_call(
        _filter_body,
        grid=(E // EB,),
        in_specs=in_specs,
        out_specs=[pl.BlockSpec((EB, H), lambda i: (i, 0))] * 3,
        out_shape=[out_sd] * 3,
    )(*args)


NB = 512


def _mm_body(x_ref, w_ref, o_ref):
    o_ref[...] = jnp.dot(x_ref[...], w_ref[...],
                         preferred_element_type=jnp.float32)


def _matmul(x, w):
    return pl.pallas_call(
        _mm_body,
        grid=(NPAD // NB,),
        in_specs=[pl.BlockSpec((NB, H), lambda i: (i, 0)),
                  pl.BlockSpec((H, H), lambda i: (0, 0))],
        out_specs=pl.BlockSpec((NB, H), lambda i: (i, 0)),
        out_shape=jax.ShapeDtypeStruct((NPAD, H), jnp.float32),
    )(x, w)


def _node_body(p0_ref, p1_ref, v_ref, w1_ref, b1_ref, w2_ref, b2_ref, o_ref):
    agg = p0_ref[...] + p1_ref[...]
    t = _ssp(jnp.dot(agg, w1_ref[...], preferred_element_type=jnp.float32)
             + b1_ref[...])
    o_ref[...] = (v_ref[...]
                  + jnp.dot(t, w2_ref[...], preferred_element_type=jnp.float32)
                  + b2_ref[...])


def _node(p0, p1, v, w1, b1, w2, b2):
    blk = pl.BlockSpec((NB, H), lambda i: (i, 0))
    full = lambda shape: pl.BlockSpec(shape, lambda i: (0, 0))
    return pl.pallas_call(
        _node_body,
        grid=(NPAD // NB,),
        in_specs=[blk, blk, blk, full((H, H)), full((1, H)),
                  full((H, H)), full((1, H))],
        out_specs=blk,
        out_shape=jax.ShapeDtypeStruct((NPAD, H), jnp.float32),
    )(p0, p1, v, w1, b1, w2, b2)


def _final_body(v_ref, batch_ref, u1_ref, b1_ref, u2_ref, b2_ref, o_ref):
    i = pl.program_id(0)

    @pl.when(i == 0)
    def _():
        o_ref[...] = jnp.zeros_like(o_ref)

    t = _ssp(jnp.dot(v_ref[...], u1_ref[...],
                     preferred_element_type=jnp.float32) + b1_ref[...])
    h = jnp.dot(t, u2_ref[...], preferred_element_type=jnp.float32) + b2_ref[...]
    rows = lax.broadcasted_iota(jnp.int32, (H, 1), 0)
    obt = (rows == batch_ref[...]).astype(jnp.float32)
    o_ref[...] += jnp.dot(obt, h, preferred_element_type=jnp.float32)


def _final(v, batch_row, u1p, b1p, u2p, b2r):
    full = lambda shape: pl.BlockSpec(shape, lambda i: (0, 0))
    return pl.pallas_call(
        _final_body,
        grid=(NPAD // NB,),
        in_specs=[pl.BlockSpec((NB, H), lambda i: (i, 0)),
                  pl.BlockSpec((1, NB), lambda i: (0, i)),
                  full((H, H)), full((1, H)), full((H, H)), full((1, H))],
        out_specs=full((H, H)),
        out_shape=jax.ShapeDtypeStruct((H, H), jnp.float32),
    )(v, batch_row, u1p, b1p, u2p, b2r)


def kernel(z, pos, batch, edge_index, params):
    row = edge_index[0].astype(jnp.int32)
    col = edge_index[1].astype(jnp.int32)
    zp = jnp.concatenate([z.astype(jnp.int32),
                          jnp.zeros((NPAD - N,), jnp.int32)])
    batchp = jnp.concatenate([batch.astype(jnp.int32),
                              jnp.full((NPAD - N,), H - 1, jnp.int32)])
    batch_row = batchp.reshape(1, NPAD)

    v0, d2 = _prep()(zp, pos.astype(jnp.float32).reshape(-1), row, col,
                   params['emb_table'].astype(jnp.float32))
    d2c = d2.reshape(E, 1)

    offs = jnp.zeros((1, H), jnp.float32).at[0, :NG].set(
        jnp.asarray(_offset_np))
    layer_ws = []
    for l in range(L):
        p = params['layers'][l]
        w0p = jnp.zeros((H, H), jnp.float32).at[:NG, :].set(p['e_mlp_W0'])
        layer_ws.append((w0p, p['e_mlp_b0'].reshape(1, H),
                         p['e_mlp_W1'], p['e_mlp_b1'].reshape(1, H)))
    w_filters = _filter(d2c, offs, layer_ws)

    v = v0
    for l in range(L):
        p = params['layers'][l]
        vv = _matmul(v, p['e_lin_W'])
        parts = _edge()(vv, w_filters[l], row, col)
        v = _node(parts[:NPAD], parts[NPAD:], v,
                  p['v_lin1_W'], p['v_lin1_b'].reshape(1, H),
                  p['v_lin2_W'], p['v_lin2_b'].reshape(1, H))

    u1p = jnp.zeros((H, H), jnp.float32).at[:, :H // 2].set(params['u_lin1_W'])
    b1p = jnp.zeros((1, H), jnp.float32).at[0, :H // 2].set(params['u_lin1_b'])
    u2p = jnp.zeros((H, H), jnp.float32).at[:H // 2, 0].set(
        params['u_lin2_W'][:, 0])
    b2r = jnp.broadcast_to(params['u_lin2_b'].reshape(1, 1), (1, H))

    u_full = _final(v, batch_row, u1p, b1p, u2p, b2r)
    u = u_full[:B, :1]
    return (u, 0.0, True, 69.0)


# speedup vs baseline: 3.0671x; 1.3530x over previous
import functools

import jax
import jax.numpy as jnp
import numpy as np
from jax import lax
from jax.experimental import pallas as pl
from jax.experimental.pallas import tpu as pltpu
from jax.experimental.pallas import tpu_sc as plsc

N = 10000
E = 320000
H = 128
NF = 128
NG = 50
B = 100
L = 3
CUTOFF = 10.0
PI = float(np.pi)

NPAD = 10240
NW = 32
EPW = E // NW
ECH = 80
NCHUNKS = EPW // ECH
NPW = NPAD // NW
NROWS_PER_SUB = NPAD // 16

_offset_np = np.linspace(0.0, CUTOFF, NG).astype(np.float32)
_COEFF = float(-0.5 / (_offset_np[1] - _offset_np[0]) ** 2)
_SHIFT = float(np.log(2.0))


def _ssp(x):
    return jnp.maximum(x, 0.0) + jnp.log1p(jnp.exp(-jnp.abs(x))) - _SHIFT


def _mesh():
    return plsc.VectorSubcoreMesh(core_axis_name="c", subcore_axis_name="s",
                                  num_cores=2, num_subcores=16)


def _prep_body(z_hbm, pos_hbm, row_hbm, col_hbm, table_hbm,
               v0_hbm, d2_hbm,
               z_v, rows_v, pos_v, ridx_v, cidx_v, d2_v, sem):
    c = lax.axis_index("c")
    s = lax.axis_index("s")
    wid = s * 2 + c

    nbase = wid * NPW
    pltpu.sync_copy(z_hbm.at[pl.ds(nbase, NPW)], z_v)
    for ci in range(NPW // ECH):
        idx = z_v.at[pl.ds(ci * ECH, ECH)]
        pltpu.async_copy(table_hbm.at[idx], rows_v, sem).wait()
        pltpu.sync_copy(rows_v, v0_hbm.at[pl.ds(nbase + ci * ECH, ECH)])

    ebase = wid * EPW
    pltpu.sync_copy(pos_hbm, pos_v)
    pltpu.sync_copy(row_hbm.at[pl.ds(ebase, EPW)], ridx_v)
    pltpu.sync_copy(col_hbm.at[pl.ds(ebase, EPW)], cidx_v)

    def body(i, carry):
        r3 = ridx_v[pl.ds(i * 16, 16)] * 3
        c3 = cidx_v[pl.ds(i * 16, 16)] * 3
        acc = jnp.zeros((16,), jnp.float32)
        for j in range(3):
            pr = plsc.load_gather(pos_v, [r3 + j])
            pc = plsc.load_gather(pos_v, [c3 + j])
            d = pr - pc
            acc = acc + d * d
        d2_v[pl.ds(i * 16, 16)] = acc
        return carry

    lax.fori_loop(0, EPW // 16, body, 0)
    pltpu.sync_copy(d2_v, d2_hbm.at[pl.ds(ebase, EPW)])


@functools.lru_cache(maxsize=None)
def _prep():
  return functools.partial(
    pl.kernel,
    out_type=(jax.ShapeDtypeStruct((NPAD, H), jnp.float32),
              jax.ShapeDtypeStruct((E,), jnp.float32)),
    mesh=_mesh(),
    scratch_types=[
        pltpu.VMEM((NPW,), jnp.int32),
        pltpu.VMEM((ECH, H), jnp.float32),
        pltpu.VMEM((N * 3,), jnp.float32),
        pltpu.VMEM((EPW,), jnp.int32),
        pltpu.VMEM((EPW,), jnp.int32),
        pltpu.VMEM((EPW,), jnp.float32),
        pltpu.SemaphoreType.DMA,
    ],
    compiler_params=pltpu.CompilerParams(needs_layout_passes=False),
  )(_prep_body)


def _edge_body(vv_hbm, w_hbm, row_hbm, col_hbm,
               out_hbm,
               acc_sm,
               ridx0, cidx0, w0, vvr0, ridx1, cidx1, w1, vvr1,
               sw0, sg0, sw1, sg1):
    c = lax.axis_index("c")
    s = lax.axis_index("s")
    wid = s * 2 + c

    zero16 = jnp.zeros((16,), jnp.float32)

    def zbody(e, carry):
        for j in range(H // 16):
            vvr0[e, pl.ds(j * 16, 16)] = zero16
        return carry

    lax.fori_loop(0, ECH, zbody, 0)
    for b in range(NROWS_PER_SUB // ECH):
        pltpu.sync_copy(vvr0, acc_sm.at[pl.ds(s * NROWS_PER_SUB + b * ECH, ECH)])
    plsc.subcore_barrier()

    def start_chunk(i, rv, cv, wv, vv, sw, sg):
        base = wid * EPW + i * ECH
        pltpu.sync_copy(row_hbm.at[pl.ds(base, ECH)], rv)
        pltpu.sync_copy(col_hbm.at[pl.ds(base, ECH)], cv)
        pltpu.async_copy(w_hbm.at[pl.ds(base, ECH)], wv, sw)
        pltpu.async_copy(vv_hbm.at[rv], vv, sg)

    def finish_chunk(i, rv, cv, wv, vv, sw, sg):
        base = wid * EPW + i * ECH
        pltpu.make_async_copy(w_hbm.at[pl.ds(base, ECH)], wv, sw).wait()
        pltpu.make_async_copy(vv_hbm.at[rv], vv, sg).wait()

        def mbody(e4, carry2):
            e = e4 * 4
            for u in range(4):
                for j in range(H // 16):
                    sl = pl.ds(j * 16, 16)
                    wv[e + u, sl] = wv[e + u, sl] * vv[e + u, sl]
            return carry2

        lax.fori_loop(0, ECH // 4, mbody, 0)
        pltpu.sync_copy(wv, acc_sm.at[cv], add=True)

    buf0 = (ridx0, cidx0, w0, vvr0, sw0, sg0)
    buf1 = (ridx1, cidx1, w1, vvr1, sw1, sg1)
    start_chunk(0, *buf0)

    def pair(k, carry):
        start_chunk(2 * k + 1, *buf1)
        finish_chunk(2 * k, *buf0)
        start_chunk(2 * k + 2, *buf0)
        finish_chunk(2 * k + 1, *buf1)
        return carry

    lax.fori_loop(0, (NCHUNKS - 1) // 2, pair, 0)
    finish_chunk(NCHUNKS - 1, *buf0)

    plsc.subcore_barrier()
    pltpu.sync_copy(acc_sm.at[pl.ds(s * NROWS_PER_SUB, NROWS_PER_SUB)],
                    out_hbm.at[pl.ds(c * NPAD + s * NROWS_PER_SUB, NROWS_PER_SUB)])


@functools.lru_cache(maxsize=None)
def _edge():
  return functools.partial(
    pl.kernel,
    out_type=jax.ShapeDtypeStruct((2 * NPAD, H), jnp.float32),
    mesh=_mesh(),
    scratch_types=[
        pltpu.VMEM_SHARED((NPAD, H), jnp.float32),
        pltpu.VMEM((ECH,), jnp.int32),
        pltpu.VMEM((ECH,), jnp.int32),
        pltpu.VMEM((ECH, H), jnp.float32),
        pltpu.VMEM((ECH, H), jnp.float32),
        pltpu.VMEM((ECH,), jnp.int32),
        pltpu.VMEM((ECH,), jnp.int32),
        pltpu.VMEM((ECH, H), jnp.float32),
        pltpu.VMEM((ECH, H), jnp.float32),
        pltpu.SemaphoreType.DMA,
        pltpu.SemaphoreType.DMA,
        pltpu.SemaphoreType.DMA,
        pltpu.SemaphoreType.DMA,
    ],
    compiler_params=pltpu.CompilerParams(needs_layout_passes=False),
  )(_edge_body)


EB = 512


def _filter_body(d2_ref, offs_ref,
                 w00, b00, w10, b10,
                 w01, b01, w11, b11,
                 w02, b02, w12, b12,
                 o0, o1, o2):
    d2 = d2_ref[...]
    dist = jnp.sqrt(d2)
    delta = dist - offs_ref[...]
    demb = jnp.exp(_COEFF * delta * delta)
    cc = 0.5 * (jnp.cos(dist * (PI / CUTOFF)) + 1.0)
    for w0r, b0r, w1r, b1r, o in ((w00, b00, w10, b10, o0),
                                  (w01, b01, w11, b11, o1),
                                  (w02, b02, w12, b12, o2)):
        h = jnp.dot(demb, w0r[...], preferred_element_type=jnp.float32) + b0r[...]
        h = _ssp(h)
        w = jnp.dot(h, w1r[...], preferred_element_type=jnp.float32) + b1r[...]
        o[...] = w * cc


def _filter(d2c, offs, layer_ws):
    full = lambda shape: pl.BlockSpec(shape, lambda i: (0, 0))
    in_specs = [pl.BlockSpec((EB, 1), lambda i: (i, 0)), full((1, H))]
    args = [d2c, offs]
    for (w0p, b0, w1, b1) in layer_ws:
        in_specs += [full((H, H)), full((1, H)), full((H, H)), full((1, H))]
        args += [w0p, b0, w1, b1]
    out_sd = jax.ShapeDtypeStruct((E, H), jnp.float32)
    return pl.pallas_call(
        _filter_body,
        grid=(E // EB,),
        in_specs=in_specs,
        out_specs=[pl.BlockSpec((EB, H), lambda i: (i, 0))] * 3,
        out_shape=[out_sd] * 3,
    )(*args)


NB = 512


def _mm_body(x_ref, w_ref, o_ref):
    o_ref[...] = jnp.dot(x_ref[...], w_ref[...],
                         preferred_element_type=jnp.float32)


def _matmul(x, w):
    return pl.pallas_call(
        _mm_body,
        grid=(NPAD // NB,),
        in_specs=[pl.BlockSpec((NB, H), lambda i: (i, 0)),
                  pl.BlockSpec((H, H), lambda i: (0, 0))],
        out_specs=pl.BlockSpec((NB, H), lambda i: (i, 0)),
        out_shape=jax.ShapeDtypeStruct((NPAD, H), jnp.float32),
    )(x, w)


def _node_body(p0_ref, p1_ref, v_ref, w1_ref, b1_ref, w2_ref, b2_ref, o_ref):
    agg = p0_ref[...] + p1_ref[...]
    t = _ssp(jnp.dot(agg, w1_ref[...], preferred_element_type=jnp.float32)
             + b1_ref[...])
    o_ref[...] = (v_ref[...]
                  + jnp.dot(t, w2_ref[...], preferred_element_type=jnp.float32)
                  + b2_ref[...])


def _node(p0, p1, v, w1, b1, w2, b2):
    blk = pl.BlockSpec((NB, H), lambda i: (i, 0))
    full = lambda shape: pl.BlockSpec(shape, lambda i: (0, 0))
    return pl.pallas_call(
        _node_body,
        grid=(NPAD // NB,),
        in_specs=[blk, blk, blk, full((H, H)), full((1, H)),
                  full((H, H)), full((1, H))],
        out_specs=blk,
        out_shape=jax.ShapeDtypeStruct((NPAD, H), jnp.float32),
    )(p0, p1, v, w1, b1, w2, b2)


def _final_body(v_ref, batch_ref, u1_ref, b1_ref, u2_ref, b2_ref, o_ref):
    i = pl.program_id(0)

    @pl.when(i == 0)
    def _():
        o_ref[...] = jnp.zeros_like(o_ref)

    t = _ssp(jnp.dot(v_ref[...], u1_ref[...],
                     preferred_element_type=jnp.float32) + b1_ref[...])
    h = jnp.dot(t, u2_ref[...], preferred_element_type=jnp.float32) + b2_ref[...]
    rows = lax.broadcasted_iota(jnp.int32, (H, 1), 0)
    obt = (rows == batch_ref[...]).astype(jnp.float32)
    o_ref[...] += jnp.dot(obt, h, preferred_element_type=jnp.float32)


def _final(v, batch_row, u1p, b1p, u2p, b2r):
    full = lambda shape: pl.BlockSpec(shape, lambda i: (0, 0))
    return pl.pallas_call(
        _final_body,
        grid=(NPAD // NB,),
        in_specs=[pl.BlockSpec((NB, H), lambda i: (i, 0)),
                  pl.BlockSpec((1, NB), lambda i: (0, i)),
                  full((H, H)), full((1, H)), full((H, H)), full((1, H))],
        out_specs=full((H, H)),
        out_shape=jax.ShapeDtypeStruct((H, H), jnp.float32),
    )(v, batch_row, u1p, b1p, u2p, b2r)


def kernel(z, pos, batch, edge_index, params):
    row = edge_index[0].astype(jnp.int32)
    col = edge_index[1].astype(jnp.int32)
    zp = jnp.concatenate([z.astype(jnp.int32),
                          jnp.zeros((NPAD - N,), jnp.int32)])
    batchp = jnp.concatenate([batch.astype(jnp.int32),
                              jnp.full((NPAD - N,), H - 1, jnp.int32)])
    batch_row = batchp.reshape(1, NPAD)

    v0, d2 = _prep()(zp, pos.astype(jnp.float32).reshape(-1), row, col,
                   params['emb_table'].astype(jnp.float32))
    d2c = d2.reshape(E, 1)

    offs = jnp.zeros((1, H), jnp.float32).at[0, :NG].set(
        jnp.asarray(_offset_np))
    layer_ws = []
    for l in range(L):
        p = params['layers'][l]
        w0p = jnp.zeros((H, H), jnp.float32).at[:NG, :].set(p['e_mlp_W0'])
        layer_ws.append((w0p, p['e_mlp_b0'].reshape(1, H),
                         p['e_mlp_W1'], p['e_mlp_b1'].reshape(1, H)))
    w_filters = _filter(d2c, offs, layer_ws)

    v = v0
    for l in range(L):
        p = params['layers'][l]
        vv = _matmul(v, p['e_lin_W'])
        parts = _edge()(vv, w_filters[l], row, col)
        v = _node(parts[:NPAD], parts[NPAD:], v,
                  p['v_lin1_W'], p['v_lin1_b'].reshape(1, H),
                  p['v_lin2_W'], p['v_lin2_b'].reshape(1, H))

    u1p = jnp.zeros((H, H), jnp.float32).at[:, :H // 2].set(params['u_lin1_W'])
    b1p = jnp.zeros((1, H), jnp.float32).at[0, :H // 2].set(params['u_lin1_b'])
    u2p = jnp.zeros((H, H), jnp.float32).at[:H // 2, 0].set(
        params['u_lin2_W'][:, 0])
    b2r = jnp.broadcast_to(params['u_lin2_b'].reshape(1, 1), (1, H))

    u_full = _final(v, batch_row, u1p, b1p, u2p, b2r)
    u = u_full[:B, :1]
    return (u, 0.0, True, 69.0)
